# baseline (speedup 1.0000x reference)
import functools

import jax
import jax.numpy as jnp
from jax import lax
from jax.experimental import pallas as pl
from jax.experimental.pallas import tpu as pltpu
from jax.experimental.pallas import tpu_sc as plsc

_XROWS = 16384
_SEQ = 50
_D = 32
_RCHUNK = 8


def _make_gather():
    info = plsc.get_sparse_core_info()
    nw = info.num_cores * info.num_subcores
    rows_per_w = _XROWS // nw
    iters = rows_per_w // _RCHUNK
    nidx = _RCHUNK * _SEQ

    mesh = plsc.VectorSubcoreMesh(core_axis_name="c", subcore_axis_name="s")

    @functools.partial(
        pl.kernel,
        mesh=mesh,
        out_type=jax.ShapeDtypeStruct((_XROWS, _SEQ, _D), jnp.float32),
        scratch_types=[
            pltpu.VMEM((_RCHUNK, _SEQ), jnp.int32),
            pltpu.VMEM((_RCHUNK, _SEQ), jnp.int32),
            pltpu.VMEM((nidx, 128), jnp.float32),
            pltpu.VMEM((nidx, _D), jnp.float32),
            pltpu.SemaphoreType.DMA,
            pltpu.SemaphoreType.DMA,
        ],
        compiler_params=pltpu.CompilerParams(needs_layout_passes=False),
    )
    def gather(x_hbm, table_hbm, out_hbm, idx_v, idx2_v, rows4_v,
               compact_v, gsem, wsem):
        wid = lax.axis_index("s") * info.num_cores + lax.axis_index("c")
        base = wid * rows_per_w

        spans = (0, 16, 32, _SEQ - 16)

        def body(i, carry):
            r0 = base + i * _RCHUNK
            pltpu.sync_copy(x_hbm.at[pl.ds(r0, _RCHUNK)], idx_v)
            for r in range(_RCHUNK):
                for s in spans:
                    v = idx_v[r, pl.ds(s, 16)]
                    idx2_v[r, pl.ds(s, 16)] = v >> 2
            gathers = [
                pltpu.async_copy(
                    table_hbm.at[idx2_v.at[r]],
                    rows4_v.at[pl.ds(r * _SEQ, _SEQ)],
                    gsem,
                )
                for r in range(_RCHUNK)
            ]
            for g in gathers:
                g.wait()

            lanes = lax.iota(jnp.int32, 16)

            def extract(r, carry2):
                for k0 in spans:
                    offs = (idx_v[r, pl.ds(k0, 16)] & 3) * _D
                    rows = r * _SEQ + k0 + lanes
                    for c in range(_D):
                        vals = plsc.load_gather(rows4_v, [rows, offs + c])
                        plsc.store_scatter(compact_v, [rows, lanes * 0 + c], vals)
                return carry2

            lax.fori_loop(0, _RCHUNK, extract, 0)

            writes = [
                pltpu.async_copy(
                    compact_v.at[pl.ds(r * _SEQ, _SEQ)],
                    out_hbm.at[r0 + r],
                    wsem,
                )
                for r in range(_RCHUNK)
            ]
            for w in writes:
                w.wait()
            return carry

        lax.fori_loop(0, iters, body, 0)

    return gather


def kernel(x, embed):
    table128 = embed.reshape(embed.shape[0] * _D // 128, 128)
    return _make_gather()(x, table128)


# speedup vs baseline: 2.2928x; 2.2928x over previous
import functools

import jax
import jax.numpy as jnp
from jax import lax
from jax.experimental import pallas as pl
from jax.experimental.pallas import tpu as pltpu
from jax.experimental.pallas import tpu_sc as plsc

_XROWS = 16384
_SEQ = 50
_D = 32
_RCHUNK = 64


def _make_gather():
    info = plsc.get_sparse_core_info()
    nw = info.num_cores * info.num_subcores
    rows_per_w = _XROWS // nw
    iters = rows_per_w // _RCHUNK

    mesh = plsc.VectorSubcoreMesh(core_axis_name="c", subcore_axis_name="s")

    @functools.partial(
        pl.kernel,
        mesh=mesh,
        out_type=jax.ShapeDtypeStruct((_XROWS, _SEQ, _D), jnp.float32),
        scratch_types=[
            pltpu.VMEM((_RCHUNK, _SEQ), jnp.int32),
            pltpu.VMEM((_RCHUNK, _SEQ, _D), jnp.float32),
            pltpu.SemaphoreType.DMA,
            pltpu.SemaphoreType.DMA,
        ],
        compiler_params=pltpu.CompilerParams(use_tc_tiling_on_sc=False),
    )
    def gather(x_hbm, table_hbm, out_hbm, idx_v, rows_v, gsem, wsem):
        wid = lax.axis_index("s") * info.num_cores + lax.axis_index("c")
        base = wid * rows_per_w

        def body(i, carry):
            r0 = base + i * _RCHUNK
            pltpu.sync_copy(x_hbm.at[pl.ds(r0, _RCHUNK)], idx_v)
            gathers = [
                pltpu.async_copy(
                    table_hbm.at[idx_v.at[r]],
                    rows_v.at[r],
                    gsem,
                )
                for r in range(_RCHUNK)
            ]
            for g in gathers:
                g.wait()
            pltpu.async_copy(
                rows_v, out_hbm.at[pl.ds(r0, _RCHUNK)], wsem
            ).wait()
            return carry

        lax.fori_loop(0, iters, body, 0)

    return gather


def kernel(x, embed):
    return _make_gather()(x, embed)


# speedup vs baseline: 2.3014x; 1.0037x over previous
import functools

import jax
import jax.numpy as jnp
from jax import lax
from jax.experimental import pallas as pl
from jax.experimental.pallas import tpu as pltpu
from jax.experimental.pallas import tpu_sc as plsc

_XROWS = 16384
_SEQ = 50
_D = 32
_RCHUNK = 64


def _make_gather():
    info = plsc.get_sparse_core_info()
    nw = info.num_cores * info.num_subcores
    rows_per_w = _XROWS // nw
    iters = rows_per_w // _RCHUNK

    mesh = plsc.VectorSubcoreMesh(core_axis_name="c", subcore_axis_name="s")

    @functools.partial(
        pl.kernel,
        mesh=mesh,
        out_type=jax.ShapeDtypeStruct((_XROWS, _SEQ, _D), jnp.float32),
        scratch_types=[
            pltpu.VMEM((_RCHUNK, _SEQ), jnp.int32),
            pltpu.VMEM((_RCHUNK, _SEQ, _D), jnp.float32),
            pltpu.SemaphoreType.DMA,
            pltpu.SemaphoreType.DMA,
        ],
        compiler_params=pltpu.CompilerParams(
            use_tc_tiling_on_sc=False,
            allow_input_fusion=(True, True),
        ),
    )
    def gather(x_hbm, table_hbm, out_hbm, idx_v, rows_v, gsem, wsem):
        wid = lax.axis_index("s") * info.num_cores + lax.axis_index("c")
        base = wid * rows_per_w

        def body(i, carry):
            r0 = base + i * _RCHUNK
            pltpu.sync_copy(x_hbm.at[pl.ds(r0, _RCHUNK)], idx_v)
            gathers = [
                pltpu.async_copy(
                    table_hbm.at[idx_v.at[r]],
                    rows_v.at[r],
                    gsem,
                )
                for r in range(_RCHUNK)
            ]
            for g in gathers:
                g.wait()
            pltpu.async_copy(
                rows_v, out_hbm.at[pl.ds(r0, _RCHUNK)], wsem
            ).wait()
            return carry

        lax.fori_loop(0, iters, body, 0)

    return gather


def kernel(x, embed):
    return _make_gather()(x, embed)
